# Initial kernel scaffold; baseline (speedup 1.0000x reference)
import jax
import jax.numpy as jnp
from jax.experimental import pallas as pl


def kernel(S_node_C4, final_S_node, H_nodes, O_nodes, H_O_edges, W_msg_node, b_msg_node, W_msg_edge, b_msg_edge, W_l1, b_l1, W_l2, b_l2, Wih_H, Whh_H, bih_H, bhh_H, Wih_S, Whh_S, bih_S, bhh_S):
    raise NotImplementedError("write your pallas kernel here")


# speedup vs baseline: 39.9712x; 39.9712x over previous
import functools

import jax
import jax.numpy as jnp
from jax.experimental import pallas as pl
from jax.experimental.pallas import tpu as pltpu

B, FM, H, O, D = 2, 8, 4, 8, 2048
HALF = D // 2
NF = B * FM
NE = NF * H * O
NH = NF * H
G3 = 3 * D


def _bdot(x, w):
    return jax.lax.dot_general(
        x.astype(jnp.bfloat16), w.astype(jnp.bfloat16),
        (((1,), (1,)), ((), ())), preferred_element_type=jnp.float32)


def _msg_body(E_ref, On_ref, Wl1_ref, Wl2_ref, Wme_ref, Wmn_ref,
              bl1_ref, bl2_ref, bme_ref, bmn_ref, out_ref):
    Omsg = _bdot(On_ref[...], Wmn_ref[...]) + bmn_ref[...]
    OmsgT = jnp.concatenate([Omsg, Omsg, Omsg, Omsg], axis=0)

    ri = jax.lax.broadcasted_iota(jnp.int32, (NE, NE), 0) // O
    ci = jax.lax.broadcasted_iota(jnp.int32, (NE, NE), 1) // O
    G8 = (ri == ci).astype(jnp.float32)

    Ecur = E_ref[...]
    UM = Ecur
    for _ in range(2):
        A = jnp.maximum(_bdot(Ecur, Wl1_ref[...]) + bl1_ref[...], 0.0)
        logit = jnp.sum(A * Wl2_ref[...], axis=1, keepdims=True) + bl2_ref[...]
        e = jnp.exp(logit - jnp.max(logit))
        gsum = jax.lax.dot_general(G8, e, (((1,), (0,)), ((), ())),
                                   preferred_element_type=jnp.float32)
        wgt = e / gsum
        Emsg = _bdot(Ecur, Wme_ref[...]) + bme_ref[...]
        UM = wgt * jnp.concatenate([Emsg, OmsgT], axis=1)
        Ecur = UM
    out_ref[...] = jnp.sum(UM.reshape(NH, O, D), axis=1) * (1.0 / O)


def _gru_h_body(Ms_ref, Hn_ref, Wih_ref, Whh_ref, bih_ref, bhh_ref,
                out_ref, r_scr, z_scr):
    i = pl.program_id(0)
    gi = _bdot(Ms_ref[...], Wih_ref[...]) + bih_ref[0]
    hn = _bdot(Hn_ref[...], Whh_ref[...]) + bhh_ref[0]
    for k in range(6):
        g, c = k // 2, k % 2
        cs = slice(c * HALF, (c + 1) * HALF)

        def _branch(g=g, cs=cs):
            if g == 0:
                r_scr[:, cs] = jax.nn.sigmoid(gi + hn)
            elif g == 1:
                z_scr[:, cs] = jax.nn.sigmoid(gi + hn)
            else:
                n = jnp.tanh(gi + r_scr[:, cs] * hn)
                z = z_scr[:, cs]
                lH = (1.0 - z) * n + z * Hn_ref[:, cs]
                out_ref[:, cs] = 0.25 * (lH[0:NF] + lH[NF:2 * NF]
                                         + lH[2 * NF:3 * NF] + lH[3 * NF:4 * NF])
        pl.when(i == k)(_branch)


def _gru_s_body(All_ref, Xh_ref, Wih_ref, Whh_ref, bih_ref, bhh_ref,
                out_ref, a_scr, b_scr, hs_scr, gh2_scr):
    i = pl.program_id(0)
    x = jnp.where(i < 6, All_ref[...], hs_scr[...])
    gi = _bdot(x, Wih_ref[...]) + bih_ref[0]
    gh = _bdot(Xh_ref[...], Whh_ref[...]) + bhh_ref[0]
    gh1 = gh[0:NF]
    gh2 = gh[NF:2 * NF]
    for k in range(12):
        phase, g, c = k // 6, (k % 6) // 2, k % 2
        cs = slice(c * HALF, (c + 1) * HALF)
        ks = slice((k % 6) * HALF, (k % 6 + 1) * HALF)

        def _branch(phase=phase, g=g, cs=cs, ks=ks):
            if phase == 0:
                gh2_scr[:, ks] = gh2
                if g == 0:
                    a_scr[:, cs] = jax.nn.sigmoid(gi + gh1)
                elif g == 1:
                    b_scr[:, cs] = jax.nn.sigmoid(gi + gh1)
                else:
                    n1 = jnp.tanh(gi + a_scr[:, cs] * gh1)
                    z1 = b_scr[:, cs]
                    hs_scr[:, cs] = (1.0 - z1) * n1 + z1 * Xh_ref[0:NF, cs]
            else:
                hn2 = gh2_scr[:, ks]
                if g == 0:
                    a_scr[:, cs] = jax.nn.sigmoid(gi + hn2)
                elif g == 1:
                    b_scr[:, cs] = jax.nn.sigmoid(gi + hn2)
                else:
                    n2 = jnp.tanh(gi + a_scr[:, cs] * hn2)
                    z2 = b_scr[:, cs]
                    out_ref[:, cs] = (1.0 - z2) * n2 + z2 * Xh_ref[NF:2 * NF, cs]
        pl.when(i == k)(_branch)


_PARAMS = pltpu.CompilerParams(dimension_semantics=("arbitrary",))


@jax.jit
def kernel(S_node_C4, final_S_node, H_nodes, O_nodes, H_O_edges,
           W_msg_node, b_msg_node, W_msg_edge, b_msg_edge,
           W_l1, b_l1, W_l2, b_l2,
           Wih_H, Whh_H, bih_H, bhh_H,
           Wih_S, Whh_S, bih_S, bhh_S):
    f32 = jnp.float32
    E0 = (H_O_edges.reshape(B, FM, H, O, D)
          .transpose(2, 0, 1, 3, 4).reshape(NE, D))
    On = O_nodes.reshape(NF * O, D)
    Hn = H_nodes.transpose(2, 0, 1, 3).reshape(NH, D)
    sC4 = S_node_C4.reshape(NF, D)
    Sf = final_S_node.transpose(0, 2, 1).reshape(NF, D)
    Xh = jnp.concatenate([sC4, Sf], axis=0)

    Msum = pl.pallas_call(
        _msg_body,
        out_shape=jax.ShapeDtypeStruct((NH, D), f32),
    )(E0, On, W_l1, W_l2, W_msg_edge, W_msg_node,
      b_l1.reshape(1, HALF), b_l2.reshape(1, 1),
      b_msg_edge.reshape(1, HALF), b_msg_node.reshape(1, HALF))

    gate_spec = pl.BlockSpec((HALF, D), lambda i: (i, 0))
    bias6 = lambda b: b.reshape(6, 1, HALF)
    bias_spec = pl.BlockSpec((1, 1, HALF), lambda i: (i % 6, 0, 0))
    full = lambda shape: pl.BlockSpec(shape, lambda i: tuple(0 for _ in shape))

    All = pl.pallas_call(
        _gru_h_body,
        grid=(6,),
        in_specs=[full((NH, D)), full((NH, D)), gate_spec, gate_spec,
                  bias_spec, bias_spec],
        out_specs=full((NF, D)),
        out_shape=jax.ShapeDtypeStruct((NF, D), f32),
        scratch_shapes=[pltpu.VMEM((NH, D), f32), pltpu.VMEM((NH, D), f32)],
        compiler_params=_PARAMS,
    )(Msum, Hn, Wih_H, Whh_H, bias6(bih_H), bias6(bhh_H))

    wih_spec = pl.BlockSpec((HALF, D), lambda i: (i % 6, 0))
    whh_spec = pl.BlockSpec((HALF, D), lambda i: (jnp.minimum(i, 5), 0))
    S_cls = pl.pallas_call(
        _gru_s_body,
        grid=(12,),
        in_specs=[full((NF, D)), full((2 * NF, D)), wih_spec, whh_spec,
                  bias_spec, bias_spec],
        out_specs=full((NF, D)),
        out_shape=jax.ShapeDtypeStruct((NF, D), f32),
        scratch_shapes=[pltpu.VMEM((NF, D), f32), pltpu.VMEM((NF, D), f32),
                        pltpu.VMEM((NF, D), f32), pltpu.VMEM((NF, G3), f32)],
        compiler_params=_PARAMS,
    )(All, Xh, Wih_S, Whh_S, bias6(bih_S), bias6(bhh_S))

    return S_cls.reshape(B, FM, D)


# speedup vs baseline: 44.9740x; 1.1252x over previous
import functools

import jax
import jax.numpy as jnp
from jax.experimental import pallas as pl
from jax.experimental.pallas import tpu as pltpu

B, FM, H, O, D = 2, 8, 4, 8, 2048
HALF = D // 2
NF = B * FM
NE = NF * H * O
NH = NF * H
G3 = 3 * D


def _bdot(x, w):
    return jax.lax.dot_general(
        x.astype(jnp.bfloat16), w.astype(jnp.bfloat16),
        (((1,), (1,)), ((), ())), preferred_element_type=jnp.float32)


def _msg_body(E_ref, On_ref, Wl1_ref, Wl2_ref, Wme_ref, Wmn_ref,
              bl1_ref, bl2_ref, bme_ref, bmn_ref, out_ref):
    Omsg = _bdot(On_ref[...], Wmn_ref[...]) + bmn_ref[...]
    OmsgT = jnp.concatenate([Omsg, Omsg, Omsg, Omsg], axis=0)

    ri = jax.lax.broadcasted_iota(jnp.int32, (NE, NE), 0) // O
    ci = jax.lax.broadcasted_iota(jnp.int32, (NE, NE), 1) // O
    G8 = (ri == ci).astype(jnp.float32)

    Ecur = E_ref[...]
    UM = Ecur
    for _ in range(2):
        A = jnp.maximum(_bdot(Ecur, Wl1_ref[...]) + bl1_ref[...], 0.0)
        logit = jnp.sum(A * Wl2_ref[...], axis=1, keepdims=True) + bl2_ref[...]
        e = jnp.exp(logit - jnp.max(logit))
        gsum = jax.lax.dot_general(G8, e, (((1,), (0,)), ((), ())),
                                   preferred_element_type=jnp.float32)
        wgt = e / gsum
        Emsg = _bdot(Ecur, Wme_ref[...]) + bme_ref[...]
        UM = wgt * jnp.concatenate([Emsg, OmsgT], axis=1)
        Ecur = UM
    out_ref[...] = jnp.sum(UM.reshape(NH, O, D), axis=1) * (1.0 / O)


def _gru_h_body(Ms_ref, Hn_ref, Wih_ref, Whh_ref, bih_ref, bhh_ref,
                out_ref, r_scr, z_scr):
    i = pl.program_id(0)
    gi = _bdot(Ms_ref[...], Wih_ref[...]) + bih_ref[0]
    hn = _bdot(Hn_ref[...], Whh_ref[...]) + bhh_ref[0]
    for k in range(6):
        g, c = k // 2, k % 2
        cs = slice(c * HALF, (c + 1) * HALF)

        def _branch(g=g, cs=cs):
            if g == 0:
                r_scr[:, cs] = jax.nn.sigmoid(gi + hn)
            elif g == 1:
                z_scr[:, cs] = jax.nn.sigmoid(gi + hn)
            else:
                n = jnp.tanh(gi + r_scr[:, cs] * hn)
                z = z_scr[:, cs]
                lH = (1.0 - z) * n + z * Hn_ref[:, cs]
                out_ref[:, cs] = 0.25 * (lH[0:NF] + lH[NF:2 * NF]
                                         + lH[2 * NF:3 * NF] + lH[3 * NF:4 * NF])
        pl.when(i == k)(_branch)


QB = 512
NQ = G3 // QB


def _gru_s_body(All_ref, Xh_ref, Wih_ref, Whh_ref, bih_ref, bhh_ref,
                out_ref, a_scr, b_scr, hs_scr, gh2_scr, wbf_scr):
    i = pl.program_id(0)
    gi = _bdot(All_ref[...], Wih_ref[...])
    gh = _bdot(Xh_ref[...], Whh_ref[...])
    for k in range(NQ):
        g, q = k * QB // D, (k * QB % D) // QB
        ks = slice(k * QB, (k + 1) * QB)
        cs = slice(q * QB, (q + 1) * QB)

        def _branch(g=g, cs=cs, ks=ks, k=k):
            wbf_scr[k * QB:(k + 1) * QB, :] = Wih_ref[...].astype(jnp.bfloat16)
            bh = bhh_ref[:, ks]
            gh2_scr[:, ks] = gh[NF:2 * NF] + bh
            g1 = gh[0:NF] + bh
            gi1 = gi + bih_ref[:, ks]
            if g == 0:
                a_scr[:, cs] = jax.nn.sigmoid(gi1 + g1)
            elif g == 1:
                b_scr[:, cs] = jax.nn.sigmoid(gi1 + g1)
            else:
                n1 = jnp.tanh(gi1 + a_scr[:, cs] * g1)
                z1 = b_scr[:, cs]
                hs_scr[:, cs] = (1.0 - z1) * n1 + z1 * Xh_ref[0:NF, cs]
        pl.when(i == k)(_branch)

    def _final():
        hs = hs_scr[...].astype(jnp.bfloat16)
        gi2 = jax.lax.dot_general(hs, wbf_scr[...], (((1,), (1,)), ((), ())),
                                  preferred_element_type=jnp.float32)
        gi2 = gi2 + bih_ref[...]
        hn2 = gh2_scr[...]
        r2 = jax.nn.sigmoid(gi2[:, 0:D] + hn2[:, 0:D])
        z2 = jax.nn.sigmoid(gi2[:, D:2 * D] + hn2[:, D:2 * D])
        n2 = jnp.tanh(gi2[:, 2 * D:] + r2 * hn2[:, 2 * D:])
        out_ref[...] = (1.0 - z2) * n2 + z2 * Xh_ref[NF:2 * NF, :]
    pl.when(i == NQ)(_final)


_PARAMS = pltpu.CompilerParams(dimension_semantics=("arbitrary",))


@jax.jit
def kernel(S_node_C4, final_S_node, H_nodes, O_nodes, H_O_edges,
           W_msg_node, b_msg_node, W_msg_edge, b_msg_edge,
           W_l1, b_l1, W_l2, b_l2,
           Wih_H, Whh_H, bih_H, bhh_H,
           Wih_S, Whh_S, bih_S, bhh_S):
    f32 = jnp.float32
    E0 = (H_O_edges.reshape(B, FM, H, O, D)
          .transpose(2, 0, 1, 3, 4).reshape(NE, D))
    On = O_nodes.reshape(NF * O, D)
    Hn = H_nodes.transpose(2, 0, 1, 3).reshape(NH, D)
    sC4 = S_node_C4.reshape(NF, D)
    Sf = final_S_node.transpose(0, 2, 1).reshape(NF, D)
    Xh = jnp.concatenate([sC4, Sf], axis=0)

    Msum = pl.pallas_call(
        _msg_body,
        out_shape=jax.ShapeDtypeStruct((NH, D), f32),
    )(E0, On, W_l1, W_l2, W_msg_edge, W_msg_node,
      b_l1.reshape(1, HALF), b_l2.reshape(1, 1),
      b_msg_edge.reshape(1, HALF), b_msg_node.reshape(1, HALF))

    gate_spec = pl.BlockSpec((HALF, D), lambda i: (i, 0))
    bias6 = lambda b: b.reshape(6, 1, HALF)
    bias_spec = pl.BlockSpec((1, 1, HALF), lambda i: (i % 6, 0, 0))
    full = lambda shape: pl.BlockSpec(shape, lambda i: tuple(0 for _ in shape))

    All = pl.pallas_call(
        _gru_h_body,
        grid=(6,),
        in_specs=[full((NH, D)), full((NH, D)), gate_spec, gate_spec,
                  bias_spec, bias_spec],
        out_specs=full((NF, D)),
        out_shape=jax.ShapeDtypeStruct((NF, D), f32),
        scratch_shapes=[pltpu.VMEM((NH, D), f32), pltpu.VMEM((NH, D), f32)],
        compiler_params=_PARAMS,
    )(Msum, Hn, Wih_H, Whh_H, bias6(bih_H), bias6(bhh_H))

    q_spec = pl.BlockSpec((QB, D), lambda i: (jnp.minimum(i, NQ - 1), 0))
    S_cls = pl.pallas_call(
        _gru_s_body,
        grid=(NQ + 1,),
        in_specs=[full((NF, D)), full((2 * NF, D)), q_spec, q_spec,
                  full((1, G3)), full((1, G3))],
        out_specs=full((NF, D)),
        out_shape=jax.ShapeDtypeStruct((NF, D), f32),
        scratch_shapes=[pltpu.VMEM((NF, D), f32), pltpu.VMEM((NF, D), f32),
                        pltpu.VMEM((NF, D), f32), pltpu.VMEM((NF, G3), f32),
                        pltpu.VMEM((G3, D), jnp.bfloat16)],
        compiler_params=_PARAMS,
    )(All, Xh, Wih_S, Whh_S, bih_S.reshape(1, G3), bhh_S.reshape(1, G3))

    return S_cls.reshape(B, FM, D)
